# baseline (speedup 1.0000x reference)
import functools

import jax
import jax.numpy as jnp
from jax import lax
from jax.experimental import pallas as pl
from jax.experimental.pallas import tpu as pltpu
from jax.experimental.pallas import tpu_sc as plsc

N = 50000
H = 128
NCHUNK = 4
CW = H // NCHUNK
NSUB = 16
NCORE = 2
NPAD = 51200
ROWS_PER_SUB = NPAD // NSUB
ZROWS = 200
EB = 128

_f32 = jnp.float32
_bf16 = jnp.bfloat16


def _conv_body(blocks_per_sub,
               src4_hbm, dst_hbm, table_hbm, out_hbm,
               sidx_q, didx_q, rows_v, zbuf_v, acc_sh,
               gs0, gs1, gs2, gs3):
    c = lax.axis_index("c")
    s = lax.axis_index("s")
    gsems = (gs0, gs1, gs2, gs3)
    n8 = blocks_per_sub // 8
    blk0 = s * blocks_per_sub

    def _zfill(i, carry):
        zbuf_v[i, :] = jnp.zeros((CW,), _bf16)
        return carry

    lax.fori_loop(0, ZROWS, _zfill, 0)

    def _load_quad(chunk, q, p):
        pltpu.sync_copy(src4_hbm.at[0, pl.ds(blk0 + 4 * q, 4), :], sidx_q.at[p])
        pltpu.sync_copy(dst_hbm.at[0, pl.ds(blk0 + 4 * q, 4), :], didx_q.at[p])

    def _gather(chunk, p, r, slot):
        pltpu.async_copy(table_hbm.at[chunk].at[sidx_q.at[p, r]],
                         rows_v.at[slot], gsems[slot])

    def _drain_scatter(p, r, slot):
        pltpu.make_async_copy(table_hbm.at[0].at[sidx_q.at[p, r]], rows_v.at[slot],
                              gsems[slot]).wait()
        pltpu.sync_copy(rows_v.at[slot], acc_sh.at[didx_q.at[p, r]], add=True)

    for k in range(NCHUNK // NCORE):
        chunk = c * (NCHUNK // NCORE) + k
        plsc.subcore_barrier()

        def _zcopy(j, carry):
            pltpu.sync_copy(zbuf_v,
                            acc_sh.at[pl.ds(s * ROWS_PER_SUB + j * ZROWS, ZROWS), :])
            return carry

        lax.fori_loop(0, ROWS_PER_SUB // ZROWS, _zcopy, 0)
        plsc.subcore_barrier()

        def _octet(i, first):
            for j in range(8):
                p, r = j // 4, j % 4
                if j == 0:
                    _load_quad(chunk, 2 * i, 0)
                if j == 4:
                    _load_quad(chunk, 2 * i + 1, 1)
                _gather(chunk, p, r, j % 4)
                uj = j - 3
                if uj >= 0:
                    _drain_scatter(uj // 4, uj % 4, uj % 4)
                elif not first:
                    _drain_scatter(1, uj + 8 - 4, (uj + 8) % 4)

        _octet(0, True)

        def _body(i, carry):
            _octet(i, False)
            return carry

        lax.fori_loop(1, n8, _body, 0)
        for r in range(1, 4):
            _drain_scatter(1, r, r)

        plsc.subcore_barrier()
        pltpu.sync_copy(acc_sh.at[pl.ds(s * ROWS_PER_SUB, ROWS_PER_SUB), :],
                        out_hbm.at[chunk, pl.ds(s * ROWS_PER_SUB, ROWS_PER_SUB), :])


def _make_conv_call(epad):
    blocks_per_sub = epad // (NSUB * EB)
    mesh = plsc.VectorSubcoreMesh(core_axis_name="c", subcore_axis_name="s")
    return pl.kernel(
        functools.partial(_conv_body, blocks_per_sub),
        out_type=jax.ShapeDtypeStruct((NCHUNK, NPAD, CW), _bf16),
        mesh=mesh,
        scratch_types=[
            pltpu.VMEM((2, 4, EB), jnp.int32),
            pltpu.VMEM((2, 4, EB), jnp.int32),
            pltpu.VMEM((4, EB, CW), _bf16),
            pltpu.VMEM((ZROWS, CW), _bf16),
            pltpu.VMEM_SHARED((NPAD, CW), _bf16),
            pltpu.SemaphoreType.DMA,
            pltpu.SemaphoreType.DMA,
            pltpu.SemaphoreType.DMA,
            pltpu.SemaphoreType.DMA,
        ],
        compiler_params=pltpu.CompilerParams(use_tc_tiling_on_sc=False),
    )


def _count_body(blocks_per_sub, e_per_sub,
                dst2_hbm, out_hbm, didx_v, ones_v, zc_v, acc_sh):
    c = lax.axis_index("c")
    s = lax.axis_index("s")

    for i in range(EB // 16):
        ones_v[pl.ds(16 * i, 16)] = jnp.ones((16,), _f32)

    def _zfill(i, carry):
        off = jnp.minimum(i * 16, ROWS_PER_SUB - 16)
        zc_v[pl.ds(off, 16)] = jnp.zeros((16,), _f32)
        return carry

    lax.fori_loop(0, (ROWS_PER_SUB + 15) // 16, _zfill, 0)

    pltpu.sync_copy(zc_v, acc_sh.at[pl.ds(s * ROWS_PER_SUB, ROWS_PER_SUB)])
    plsc.subcore_barrier()

    def _ebody(b, carry):
        base = s * e_per_sub + b * EB
        pltpu.sync_copy(dst2_hbm.at[c, 0, pl.ds(base, EB)], didx_v)
        pltpu.sync_copy(ones_v, acc_sh.at[didx_v], add=True)
        return carry

    lax.fori_loop(0, blocks_per_sub, _ebody, 0)
    plsc.subcore_barrier()
    pltpu.sync_copy(acc_sh.at[pl.ds(s * ROWS_PER_SUB, ROWS_PER_SUB)],
                    out_hbm.at[c, 0, pl.ds(s * ROWS_PER_SUB, ROWS_PER_SUB)])


def _make_count_call(epad):
    blocks_per_sub = epad // (NSUB * EB)
    e_per_sub = epad // NSUB
    mesh = plsc.VectorSubcoreMesh(core_axis_name="c", subcore_axis_name="s")
    return pl.kernel(
        functools.partial(_count_body, blocks_per_sub, e_per_sub),
        out_type=jax.ShapeDtypeStruct((NCORE, 1, NPAD), _f32),
        mesh=mesh,
        scratch_types=[
            pltpu.VMEM((EB,), jnp.int32),
            pltpu.VMEM((EB,), _f32),
            pltpu.VMEM((ROWS_PER_SUB,), _f32),
            pltpu.VMEM_SHARED((NPAD,), _f32),
        ],
        compiler_params=pltpu.CompilerParams(use_tc_tiling_on_sc=False),
    )


R = 2000
GRID = N // R

_full = lambda shape: pl.BlockSpec(shape, lambda i: tuple(0 for _ in shape))
_rows = lambda: pl.BlockSpec((R, H), lambda i: (i, 0))
_crows = lambda: pl.BlockSpec((NCHUNK, R, CW), lambda i: (0, i, 0))


def _proj_body(xu, xi, wpu, bpu, wpi, bpi, wlu, wli, hu, hi, yu, yi):
    a = jnp.dot(xu[...], wpu[...], preferred_element_type=_f32) + bpu[...]
    b = jnp.dot(xi[...], wpi[...], preferred_element_type=_f32) + bpi[...]
    hu[...] = a
    hi[...] = b
    ya = jnp.dot(a, wlu[...], preferred_element_type=_f32).astype(_bf16)
    yb = jnp.dot(b, wli[...], preferred_element_type=_f32).astype(_bf16)
    for c in range(NCHUNK):
        yu[c] = ya[:, c * CW:(c + 1) * CW]
        yi[c] = yb[:, c * CW:(c + 1) * CW]


_proj_call = pl.pallas_call(
    _proj_body,
    grid=(GRID,),
    in_specs=[_rows(), _rows(),
              _full((H, H)), _full((1, H)), _full((H, H)), _full((1, H)),
              _full((H, H)), _full((H, H))],
    out_specs=[_rows(), _rows(), _crows(), _crows()],
    out_shape=[jax.ShapeDtypeStruct((N, H), _f32),
               jax.ShapeDtypeStruct((N, H), _f32),
               jax.ShapeDtypeStruct((NCHUNK, N, CW), _bf16),
               jax.ShapeDtypeStruct((NCHUNK, N, CW), _bf16)],
)


def _sage_post(seg_ref, cnt, h, wr, bl, g, b):
    seg = jnp.concatenate([seg_ref[c] for c in range(NCHUNK)],
                          axis=-1).astype(_f32)
    recip = 1.0 / jnp.maximum(cnt, 1.0)
    m = seg * recip + bl + jnp.dot(h, wr, preferred_element_type=_f32)
    mu = jnp.mean(m, axis=-1, keepdims=True)
    var = jnp.mean(jnp.square(m - mu), axis=-1, keepdims=True)
    ln = (m - mu) * lax.rsqrt(var + 1e-5) * g + b
    return jnp.maximum(ln, 0.0) + h


def _post_body_mid(segi, segu, cnts, hu, hi,
                   wri, bli, gi, bi, wru, blu, gu, bu, wlu_n, wli_n,
                   hu_o, hi_o, yu_o, yi_o):
    ni = _sage_post(segi, cnts[0], hi[...], wri[...], bli[...], gi[...], bi[...])
    nu = _sage_post(segu, cnts[1], hu[...], wru[...], blu[...], gu[...], bu[...])
    hu_o[...] = nu
    hi_o[...] = ni
    ya = jnp.dot(nu, wlu_n[...], preferred_element_type=_f32).astype(_bf16)
    yb = jnp.dot(ni, wli_n[...], preferred_element_type=_f32).astype(_bf16)
    for c in range(NCHUNK):
        yu_o[c] = ya[:, c * CW:(c + 1) * CW]
        yi_o[c] = yb[:, c * CW:(c + 1) * CW]


def _post_body_last(segi, segu, cnts, hu, hi,
                    wri, bli, gi, bi, wru, blu, gu, bu,
                    hu_o, hi_o):
    ni = _sage_post(segi, cnts[0], hi[...], wri[...], bli[...], gi[...], bi[...])
    nu = _sage_post(segu, cnts[1], hu[...], wru[...], blu[...], gu[...], bu[...])
    hu_o[...] = nu
    hi_o[...] = ni


_seg_spec = pl.BlockSpec((NCHUNK, R, CW), lambda i: (0, i, 0))
_cnt_spec = pl.BlockSpec((NCORE, R, 1), lambda i: (0, i, 0))
_w_specs = [_full((H, H)), _full((1, H)), _full((1, H)), _full((1, H)),
            _full((H, H)), _full((1, H)), _full((1, H)), _full((1, H))]

_post_mid_call = pl.pallas_call(
    _post_body_mid,
    grid=(GRID,),
    in_specs=[_seg_spec, _seg_spec, _cnt_spec, _rows(), _rows()]
             + _w_specs + [_full((H, H)), _full((H, H))],
    out_specs=[_rows(), _rows(), _crows(), _crows()],
    out_shape=[jax.ShapeDtypeStruct((N, H), _f32),
               jax.ShapeDtypeStruct((N, H), _f32),
               jax.ShapeDtypeStruct((NCHUNK, N, CW), _bf16),
               jax.ShapeDtypeStruct((NCHUNK, N, CW), _bf16)],
)

_post_last_call = pl.pallas_call(
    _post_body_last,
    grid=(GRID,),
    in_specs=[_seg_spec, _seg_spec, _cnt_spec, _rows(), _rows()] + _w_specs,
    out_specs=[_rows(), _rows()],
    out_shape=[jax.ShapeDtypeStruct((N, H), _f32),
               jax.ShapeDtypeStruct((N, H), _f32)],
)


def kernel(x_user, x_item, edge_index_user_to_item, edge_index_item_rev_to_user, params):
    e = edge_index_user_to_item.shape[1]
    quantum = NSUB * EB * 8
    epad = ((e + quantum - 1) // quantum) * quantum
    nblk = epad // EB

    spread = jnp.arange(epad - e, dtype=jnp.int32) % 1024

    def _pad(v, base):
        return jnp.concatenate([v, base + spread])

    srcu = _pad(edge_index_user_to_item[0], 0)
    dstu = _pad(edge_index_user_to_item[1], N)
    srci = _pad(edge_index_item_rev_to_user[0], 0)
    dsti = _pad(edge_index_item_rev_to_user[1], N)
    srcu_b = srcu.reshape(1, nblk, EB)
    srci_b = srci.reshape(1, nblk, EB)
    dstu_b = dstu.reshape(1, nblk, EB)
    dsti_b = dsti.reshape(1, nblk, EB)
    dst2 = jnp.stack([dstu[None, :], dsti[None, :]])

    conv = _make_conv_call(epad)
    counts = _make_count_call(epad)(dst2)
    cnt3 = counts.reshape(NCORE, NPAD)[:, :, None]

    pu_w, pu_b = params["proj_user"]
    pi_w, pi_b = params["proj_item"]
    l0, l1 = params["layers"]

    hu, hi, yu, yi = _proj_call(
        x_user, x_item, pu_w, pu_b[None, :], pi_w, pi_b[None, :],
        l0["u2i"][0], l0["i2u"][0])

    segi = conv(srcu_b, dstu_b, yu)
    segu = conv(srci_b, dsti_b, yi)
    hu, hi, yu, yi = _post_mid_call(
        segi, segu, cnt3, hu, hi,
        l0["u2i"][2], l0["u2i"][1][None, :], l0["ln_item"][0][None, :], l0["ln_item"][1][None, :],
        l0["i2u"][2], l0["i2u"][1][None, :], l0["ln_user"][0][None, :], l0["ln_user"][1][None, :],
        l1["u2i"][0], l1["i2u"][0])

    segi = conv(srcu_b, dstu_b, yu)
    segu = conv(srci_b, dsti_b, yi)
    hu, hi = _post_last_call(
        segi, segu, cnt3, hu, hi,
        l1["u2i"][2], l1["u2i"][1][None, :], l1["ln_item"][0][None, :], l1["ln_item"][1][None, :],
        l1["i2u"][2], l1["i2u"][1][None, :], l1["ln_user"][0][None, :], l1["ln_user"][1][None, :])

    return hu, hi


# speedup vs baseline: 1.5319x; 1.5319x over previous
import functools

import jax
import jax.numpy as jnp
from jax import lax
from jax.experimental import pallas as pl
from jax.experimental.pallas import tpu as pltpu
from jax.experimental.pallas import tpu_sc as plsc

N = 50000
H = 128
NCHUNK = 2
CW = H // NCHUNK
NSUB = 16
NCORE = 2
NPAD = 51200
ROWS_PER_SUB = NPAD // NSUB
ZROWS = 200
EB = 128

_f32 = jnp.float32
_bf16 = jnp.bfloat16


def _conv_body(blocks_per_sub,
               src4_hbm, dst_hbm, table_hbm, out_hbm,
               sidx_q, didx_q, rows_v, zbuf_v, acc_sh,
               gs0, gs1, gs2, gs3):
    c = lax.axis_index("c")
    s = lax.axis_index("s")
    gsems = (gs0, gs1, gs2, gs3)
    n8 = blocks_per_sub // 8
    blk0 = s * blocks_per_sub

    def _zfill(i, carry):
        zbuf_v[i, pl.ds(0, 32)] = jnp.zeros((32,), _bf16)
        zbuf_v[i, pl.ds(32, 32)] = jnp.zeros((32,), _bf16)
        return carry

    lax.fori_loop(0, ZROWS, _zfill, 0)

    def _load_quad(chunk, q, p):
        pltpu.sync_copy(src4_hbm.at[0, pl.ds(blk0 + 4 * q, 4), :], sidx_q.at[p])
        pltpu.sync_copy(dst_hbm.at[0, pl.ds(blk0 + 4 * q, 4), :], didx_q.at[p])

    def _gather(chunk, p, r, slot):
        pltpu.async_copy(table_hbm.at[chunk].at[sidx_q.at[p, r]],
                         rows_v.at[slot], gsems[slot])

    def _drain_scatter(p, r, slot):
        pltpu.make_async_copy(table_hbm.at[0].at[sidx_q.at[p, r]], rows_v.at[slot],
                              gsems[slot]).wait()
        pltpu.sync_copy(rows_v.at[slot], acc_sh.at[didx_q.at[p, r]], add=True)

    for k in range(NCHUNK // NCORE):
        chunk = c * (NCHUNK // NCORE) + k
        plsc.subcore_barrier()

        def _zcopy(j, carry):
            pltpu.sync_copy(zbuf_v,
                            acc_sh.at[pl.ds(s * ROWS_PER_SUB + j * ZROWS, ZROWS), :])
            return carry

        lax.fori_loop(0, ROWS_PER_SUB // ZROWS, _zcopy, 0)
        plsc.subcore_barrier()

        def _octet(i, first):
            for j in range(8):
                p, r = j // 4, j % 4
                if j == 0:
                    _load_quad(chunk, 2 * i, 0)
                if j == 4:
                    _load_quad(chunk, 2 * i + 1, 1)
                _gather(chunk, p, r, j % 4)
                uj = j - 3
                if uj >= 0:
                    _drain_scatter(uj // 4, uj % 4, uj % 4)
                elif not first:
                    _drain_scatter(1, uj + 8 - 4, (uj + 8) % 4)

        _octet(0, True)

        def _body(i, carry):
            _octet(i, False)
            return carry

        lax.fori_loop(1, n8, _body, 0)
        for r in range(1, 4):
            _drain_scatter(1, r, r)

        plsc.subcore_barrier()
        pltpu.sync_copy(acc_sh.at[pl.ds(s * ROWS_PER_SUB, ROWS_PER_SUB), :],
                        out_hbm.at[chunk, pl.ds(s * ROWS_PER_SUB, ROWS_PER_SUB), :])


def _make_conv_call(epad):
    blocks_per_sub = epad // (NSUB * EB)
    mesh = plsc.VectorSubcoreMesh(core_axis_name="c", subcore_axis_name="s")
    return pl.kernel(
        functools.partial(_conv_body, blocks_per_sub),
        out_type=jax.ShapeDtypeStruct((NCHUNK, NPAD, CW), _bf16),
        mesh=mesh,
        scratch_types=[
            pltpu.VMEM((2, 4, EB), jnp.int32),
            pltpu.VMEM((2, 4, EB), jnp.int32),
            pltpu.VMEM((4, EB, CW), _bf16),
            pltpu.VMEM((ZROWS, CW), _bf16),
            pltpu.VMEM_SHARED((NPAD, CW), _bf16),
            pltpu.SemaphoreType.DMA,
            pltpu.SemaphoreType.DMA,
            pltpu.SemaphoreType.DMA,
            pltpu.SemaphoreType.DMA,
        ],
        compiler_params=pltpu.CompilerParams(use_tc_tiling_on_sc=False),
    )


def _count_body(blocks_per_sub, e_per_sub,
                dst2_hbm, out_hbm, didx_v, ones_v, zc_v, acc_sh):
    c = lax.axis_index("c")
    s = lax.axis_index("s")

    for i in range(EB // 16):
        ones_v[pl.ds(16 * i, 16)] = jnp.ones((16,), _f32)

    def _zfill(i, carry):
        off = jnp.minimum(i * 16, ROWS_PER_SUB - 16)
        zc_v[pl.ds(off, 16)] = jnp.zeros((16,), _f32)
        return carry

    lax.fori_loop(0, (ROWS_PER_SUB + 15) // 16, _zfill, 0)

    pltpu.sync_copy(zc_v, acc_sh.at[pl.ds(s * ROWS_PER_SUB, ROWS_PER_SUB)])
    plsc.subcore_barrier()

    def _ebody(b, carry):
        base = s * e_per_sub + b * EB
        pltpu.sync_copy(dst2_hbm.at[c, 0, pl.ds(base, EB)], didx_v)
        pltpu.sync_copy(ones_v, acc_sh.at[didx_v], add=True)
        return carry

    lax.fori_loop(0, blocks_per_sub, _ebody, 0)
    plsc.subcore_barrier()
    pltpu.sync_copy(acc_sh.at[pl.ds(s * ROWS_PER_SUB, ROWS_PER_SUB)],
                    out_hbm.at[c, 0, pl.ds(s * ROWS_PER_SUB, ROWS_PER_SUB)])


def _make_count_call(epad):
    blocks_per_sub = epad // (NSUB * EB)
    e_per_sub = epad // NSUB
    mesh = plsc.VectorSubcoreMesh(core_axis_name="c", subcore_axis_name="s")
    return pl.kernel(
        functools.partial(_count_body, blocks_per_sub, e_per_sub),
        out_type=jax.ShapeDtypeStruct((NCORE, 1, NPAD), _f32),
        mesh=mesh,
        scratch_types=[
            pltpu.VMEM((EB,), jnp.int32),
            pltpu.VMEM((EB,), _f32),
            pltpu.VMEM((ROWS_PER_SUB,), _f32),
            pltpu.VMEM_SHARED((NPAD,), _f32),
        ],
        compiler_params=pltpu.CompilerParams(use_tc_tiling_on_sc=False),
    )


R = 2000
GRID = N // R

_full = lambda shape: pl.BlockSpec(shape, lambda i: tuple(0 for _ in shape))
_rows = lambda: pl.BlockSpec((R, H), lambda i: (i, 0))
_crows = lambda: pl.BlockSpec((NCHUNK, R, CW), lambda i: (0, i, 0))


def _proj_body(xu, xi, wpu, bpu, wpi, bpi, wlu, wli, hu, hi, yu, yi):
    a = jnp.dot(xu[...], wpu[...], preferred_element_type=_f32) + bpu[...]
    b = jnp.dot(xi[...], wpi[...], preferred_element_type=_f32) + bpi[...]
    hu[...] = a
    hi[...] = b
    ya = jnp.dot(a, wlu[...], preferred_element_type=_f32).astype(_bf16)
    yb = jnp.dot(b, wli[...], preferred_element_type=_f32).astype(_bf16)
    for c in range(NCHUNK):
        yu[c] = ya[:, c * CW:(c + 1) * CW]
        yi[c] = yb[:, c * CW:(c + 1) * CW]


_proj_call = pl.pallas_call(
    _proj_body,
    grid=(GRID,),
    in_specs=[_rows(), _rows(),
              _full((H, H)), _full((1, H)), _full((H, H)), _full((1, H)),
              _full((H, H)), _full((H, H))],
    out_specs=[_rows(), _rows(), _crows(), _crows()],
    out_shape=[jax.ShapeDtypeStruct((N, H), _f32),
               jax.ShapeDtypeStruct((N, H), _f32),
               jax.ShapeDtypeStruct((NCHUNK, N, CW), _bf16),
               jax.ShapeDtypeStruct((NCHUNK, N, CW), _bf16)],
)


def _sage_post(seg_ref, cnt, h, wr, bl, g, b):
    seg = jnp.concatenate([seg_ref[c] for c in range(NCHUNK)],
                          axis=-1).astype(_f32)
    recip = 1.0 / jnp.maximum(cnt, 1.0)
    m = seg * recip + bl + jnp.dot(h, wr, preferred_element_type=_f32)
    mu = jnp.mean(m, axis=-1, keepdims=True)
    var = jnp.mean(jnp.square(m - mu), axis=-1, keepdims=True)
    ln = (m - mu) * lax.rsqrt(var + 1e-5) * g + b
    return jnp.maximum(ln, 0.0) + h


def _post_body_mid(segi, segu, cnts, hu, hi,
                   wri, bli, gi, bi, wru, blu, gu, bu, wlu_n, wli_n,
                   hu_o, hi_o, yu_o, yi_o):
    ni = _sage_post(segi, cnts[0], hi[...], wri[...], bli[...], gi[...], bi[...])
    nu = _sage_post(segu, cnts[1], hu[...], wru[...], blu[...], gu[...], bu[...])
    hu_o[...] = nu
    hi_o[...] = ni
    ya = jnp.dot(nu, wlu_n[...], preferred_element_type=_f32).astype(_bf16)
    yb = jnp.dot(ni, wli_n[...], preferred_element_type=_f32).astype(_bf16)
    for c in range(NCHUNK):
        yu_o[c] = ya[:, c * CW:(c + 1) * CW]
        yi_o[c] = yb[:, c * CW:(c + 1) * CW]


def _post_body_last(segi, segu, cnts, hu, hi,
                    wri, bli, gi, bi, wru, blu, gu, bu,
                    hu_o, hi_o):
    ni = _sage_post(segi, cnts[0], hi[...], wri[...], bli[...], gi[...], bi[...])
    nu = _sage_post(segu, cnts[1], hu[...], wru[...], blu[...], gu[...], bu[...])
    hu_o[...] = nu
    hi_o[...] = ni


_seg_spec = pl.BlockSpec((NCHUNK, R, CW), lambda i: (0, i, 0))
_cnt_spec = pl.BlockSpec((NCORE, R, 1), lambda i: (0, i, 0))
_w_specs = [_full((H, H)), _full((1, H)), _full((1, H)), _full((1, H)),
            _full((H, H)), _full((1, H)), _full((1, H)), _full((1, H))]

_post_mid_call = pl.pallas_call(
    _post_body_mid,
    grid=(GRID,),
    in_specs=[_seg_spec, _seg_spec, _cnt_spec, _rows(), _rows()]
             + _w_specs + [_full((H, H)), _full((H, H))],
    out_specs=[_rows(), _rows(), _crows(), _crows()],
    out_shape=[jax.ShapeDtypeStruct((N, H), _f32),
               jax.ShapeDtypeStruct((N, H), _f32),
               jax.ShapeDtypeStruct((NCHUNK, N, CW), _bf16),
               jax.ShapeDtypeStruct((NCHUNK, N, CW), _bf16)],
)

_post_last_call = pl.pallas_call(
    _post_body_last,
    grid=(GRID,),
    in_specs=[_seg_spec, _seg_spec, _cnt_spec, _rows(), _rows()] + _w_specs,
    out_specs=[_rows(), _rows()],
    out_shape=[jax.ShapeDtypeStruct((N, H), _f32),
               jax.ShapeDtypeStruct((N, H), _f32)],
)


def kernel(x_user, x_item, edge_index_user_to_item, edge_index_item_rev_to_user, params):
    e = edge_index_user_to_item.shape[1]
    quantum = NSUB * EB * 8
    epad = ((e + quantum - 1) // quantum) * quantum
    nblk = epad // EB

    spread = jnp.arange(epad - e, dtype=jnp.int32) % 1024

    def _pad(v, base):
        return jnp.concatenate([v, base + spread])

    srcu = _pad(edge_index_user_to_item[0], 0)
    dstu = _pad(edge_index_user_to_item[1], N)
    srci = _pad(edge_index_item_rev_to_user[0], 0)
    dsti = _pad(edge_index_item_rev_to_user[1], N)
    srcu_b = srcu.reshape(1, nblk, EB)
    srci_b = srci.reshape(1, nblk, EB)
    dstu_b = dstu.reshape(1, nblk, EB)
    dsti_b = dsti.reshape(1, nblk, EB)
    dst2 = jnp.stack([dstu[None, :], dsti[None, :]])

    conv = _make_conv_call(epad)
    counts = _make_count_call(epad)(dst2)
    cnt3 = counts.reshape(NCORE, NPAD)[:, :, None]

    pu_w, pu_b = params["proj_user"]
    pi_w, pi_b = params["proj_item"]
    l0, l1 = params["layers"]

    hu, hi, yu, yi = _proj_call(
        x_user, x_item, pu_w, pu_b[None, :], pi_w, pi_b[None, :],
        l0["u2i"][0], l0["i2u"][0])

    segi = conv(srcu_b, dstu_b, yu)
    segu = conv(srci_b, dsti_b, yi)
    hu, hi, yu, yi = _post_mid_call(
        segi, segu, cnt3, hu, hi,
        l0["u2i"][2], l0["u2i"][1][None, :], l0["ln_item"][0][None, :], l0["ln_item"][1][None, :],
        l0["i2u"][2], l0["i2u"][1][None, :], l0["ln_user"][0][None, :], l0["ln_user"][1][None, :],
        l1["u2i"][0], l1["i2u"][0])

    segi = conv(srcu_b, dstu_b, yu)
    segu = conv(srci_b, dsti_b, yi)
    hu, hi = _post_last_call(
        segi, segu, cnt3, hu, hi,
        l1["u2i"][2], l1["u2i"][1][None, :], l1["ln_item"][0][None, :], l1["ln_item"][1][None, :],
        l1["i2u"][2], l1["i2u"][1][None, :], l1["ln_user"][0][None, :], l1["ln_user"][1][None, :])

    return hu, hi
